# baseline (speedup 1.0000x reference)
import jax
import jax.numpy as jnp
from jax import lax
from jax.experimental import pallas as pl
from jax.experimental.pallas import tpu as pltpu
from jax.experimental.pallas import tpu_sc as plsc

NUM_EMBEDDINGS = 100000
EMBED_DIM = 128
BATCH = 4096
HIST = 200

NC = 2
NS = 16
NW = NC * NS

WORDS = EMBED_DIM // 2
STEP_ROWS = 256
N_FLAT = BATCH * HIST
ROWS_PER_W = N_FLAT // NW
STEPS = ROWS_PER_W // STEP_ROWS
P = 3
LOOKAHEAD = 2
DRAINLAG = P - LOOKAHEAD


def _gather_body(x_hbm, table_hbm, out_hbm, idx_v, rows_v, gsems, ssems):
    wid = lax.axis_index("s") * NC + lax.axis_index("c")
    base_row = wid * ROWS_PER_W
    pltpu.sync_copy(x_hbm.at[pl.ds(base_row, ROWS_PER_W)], idx_v)

    def gather_args(t, p):
        return (
            table_hbm.at[idx_v.at[pl.ds(STEP_ROWS * t, STEP_ROWS)]],
            rows_v.at[p],
            gsems.at[p],
        )

    def store_args(t, p):
        return (
            rows_v.at[p],
            out_hbm.at[pl.ds(base_row + STEP_ROWS * t, STEP_ROWS)],
            ssems.at[p],
        )

    def step(t, b, do_drain, do_fire):
        if do_drain:
            pltpu.make_async_copy(
                *store_args(t - DRAINLAG, (t - DRAINLAG) % P)
            ).wait()
        if do_fire:
            pltpu.async_copy(*gather_args(t + LOOKAHEAD, (t + LOOKAHEAD) % P))
        pltpu.make_async_copy(*gather_args(t, b)).wait()
        pltpu.async_copy(*store_args(t, b))

    for t in range(LOOKAHEAD):
        pltpu.async_copy(*gather_args(t, t % P))
    step(0, 0, do_drain=False, do_fire=True)

    def group(g, carry):
        for r in range(1, P + 1):
            t = P * g + r
            step(t, r % P, do_drain=True, do_fire=True)
        return carry

    lax.fori_loop(0, (STEPS - 4) // P, group, 0, unroll=False)

    for t in range(STEPS - 3, STEPS):
        step(t, t % P, do_drain=True, do_fire=(t + LOOKAHEAD < STEPS))
    for t in range(STEPS - DRAINLAG, STEPS):
        pltpu.make_async_copy(*store_args(t, t % P)).wait()


@jax.jit
def _embed_lookup(x, table):
    mesh = plsc.VectorSubcoreMesh(
        core_axis_name="c", subcore_axis_name="s", num_cores=NC, num_subcores=NS
    )
    run = pl.kernel(
        _gather_body,
        out_type=jax.ShapeDtypeStruct((N_FLAT, WORDS), jnp.int32),
        mesh=mesh,
        compiler_params=pltpu.CompilerParams(use_tc_tiling_on_sc=False),
        scratch_types=[
            pltpu.VMEM((ROWS_PER_W,), jnp.int32),
            pltpu.VMEM((P, STEP_ROWS, WORDS), jnp.int32),
            pltpu.SemaphoreType.DMA((P,)),
            pltpu.SemaphoreType.DMA((P,)),
        ],
    )
    x1d = x.reshape(N_FLAT)
    tb = table.astype(jnp.bfloat16)
    tp = lax.bitcast_convert_type(
        tb.reshape(NUM_EMBEDDINGS, WORDS, 2), jnp.int32
    )
    out_packed = run(x1d, tp)
    out_bf = lax.bitcast_convert_type(out_packed, jnp.bfloat16)
    return (
        out_bf.astype(jnp.float32)
        .reshape(BATCH, HIST, EMBED_DIM)
    )


def kernel(x, table):
    return _embed_lookup(x, table)


# speedup vs baseline: 1.9949x; 1.9949x over previous
import jax
import jax.numpy as jnp
import numpy as np
from jax import lax
from jax.experimental import pallas as pl
from jax.experimental.pallas import tpu as pltpu
from jax.experimental.pallas import tpu_sc as plsc

NUM_EMBEDDINGS = 100000
EMBED_DIM = 128
BATCH = 4096
HIST = 200

NC = 2
NS = 16
NW = NC * NS

WORDS = EMBED_DIM // 2
STEP_ROWS = 256
N_FLAT = BATCH * HIST
ROWS_PER_W = N_FLAT // NW
STEPS = ROWS_PER_W // STEP_ROWS
P = 3
LOOKAHEAD = 2
DRAINLAG = P - LOOKAHEAD

_PERM = np.zeros(EMBED_DIM, np.int32)
for _i in range(WORDS):
    _PERM[2 * _i] = _i
    _PERM[2 * _i + 1] = WORDS + _i


def _gather_body(x_hbm, table_hbm, out_hbm, idx_v, rows_v, gsems, ssems):
    wid = lax.axis_index("s") * NC + lax.axis_index("c")
    base_row = wid * ROWS_PER_W
    pltpu.sync_copy(x_hbm.at[pl.ds(base_row, ROWS_PER_W)], idx_v)

    def gather_args(t, p):
        return (
            table_hbm.at[idx_v.at[pl.ds(STEP_ROWS * t, STEP_ROWS)]],
            rows_v.at[p],
            gsems.at[p],
        )

    def store_args(t, p):
        return (
            rows_v.at[p],
            out_hbm.at[pl.ds(base_row + STEP_ROWS * t, STEP_ROWS)],
            ssems.at[p],
        )

    def step(t, b, do_drain, do_fire):
        if do_drain:
            pltpu.make_async_copy(
                *store_args(t - DRAINLAG, (t - DRAINLAG) % P)
            ).wait()
        if do_fire:
            pltpu.async_copy(*gather_args(t + LOOKAHEAD, (t + LOOKAHEAD) % P))
        pltpu.make_async_copy(*gather_args(t, b)).wait()
        pltpu.async_copy(*store_args(t, b))

    for t in range(LOOKAHEAD):
        pltpu.async_copy(*gather_args(t, t % P))
    step(0, 0, do_drain=False, do_fire=True)

    def group(g, carry):
        for r in range(1, P + 1):
            t = P * g + r
            step(t, r % P, do_drain=True, do_fire=True)
        return carry

    lax.fori_loop(0, (STEPS - 4) // P, group, 0, unroll=False)

    for t in range(STEPS - 3, STEPS):
        step(t, t % P, do_drain=True, do_fire=(t + LOOKAHEAD < STEPS))
    for t in range(STEPS - DRAINLAG, STEPS):
        pltpu.make_async_copy(*store_args(t, t % P)).wait()


@jax.jit
def _embed_lookup(x, table):
    mesh = plsc.VectorSubcoreMesh(
        core_axis_name="c", subcore_axis_name="s", num_cores=NC, num_subcores=NS
    )
    run = pl.kernel(
        _gather_body,
        out_type=jax.ShapeDtypeStruct((N_FLAT, WORDS), jnp.int32),
        mesh=mesh,
        compiler_params=pltpu.CompilerParams(use_tc_tiling_on_sc=False),
        scratch_types=[
            pltpu.VMEM((ROWS_PER_W,), jnp.int32),
            pltpu.VMEM((P, STEP_ROWS, WORDS), jnp.int32),
            pltpu.SemaphoreType.DMA((P,)),
            pltpu.SemaphoreType.DMA((P,)),
        ],
    )
    x1d = x.reshape(N_FLAT)
    tb = table.astype(jnp.bfloat16)[:, _PERM]
    tp = lax.bitcast_convert_type(
        tb.reshape(NUM_EMBEDDINGS, WORDS, 2), jnp.int32
    )
    out_packed = run(x1d, tp)
    lo = lax.bitcast_convert_type(out_packed << 16, jnp.float32)
    hi = lax.bitcast_convert_type(
        out_packed & jnp.int32(-65536), jnp.float32
    )
    out = jnp.concatenate([lo, hi], axis=1)
    return out.reshape(BATCH, HIST, EMBED_DIM)


def kernel(x, table):
    return _embed_lookup(x, table)


# speedup vs baseline: 11.1180x; 5.5731x over previous
import jax
import jax.numpy as jnp
from jax import lax
from jax.experimental import pallas as pl
from jax.experimental.pallas import tpu as pltpu
from jax.experimental.pallas import tpu_sc as plsc

NUM_EMBEDDINGS = 100000
EMBED_DIM = 128
BATCH = 4096
HIST = 200

NC = 2
NS = 16
NW = NC * NS

STEP_ROWS = 256
N_FLAT = BATCH * HIST
ROWS_PER_W = N_FLAT // NW
STEPS = ROWS_PER_W // STEP_ROWS
P = 3
LOOKAHEAD = 2
DRAINLAG = P - LOOKAHEAD


def _gather_body(x_hbm, table_hbm, out_hbm, idx_v, rows_v, gsems, ssems):
    wid = lax.axis_index("s") * NC + lax.axis_index("c")
    base_row = wid * ROWS_PER_W
    pltpu.sync_copy(x_hbm.at[pl.ds(base_row, ROWS_PER_W)], idx_v)

    def gather_args(t, p):
        return (
            table_hbm.at[idx_v.at[pl.ds(STEP_ROWS * t, STEP_ROWS)]],
            rows_v.at[p],
            gsems.at[p],
        )

    def store_args(t, p):
        return (
            rows_v.at[p],
            out_hbm.at[pl.ds(base_row + STEP_ROWS * t, STEP_ROWS)],
            ssems.at[p],
        )

    def step(t, b, do_drain, do_fire):
        if do_drain:
            pltpu.make_async_copy(
                *store_args(t - DRAINLAG, (t - DRAINLAG) % P)
            ).wait()
        if do_fire:
            pltpu.async_copy(*gather_args(t + LOOKAHEAD, (t + LOOKAHEAD) % P))
        pltpu.make_async_copy(*gather_args(t, b)).wait()
        pltpu.async_copy(*store_args(t, b))

    for t in range(LOOKAHEAD):
        pltpu.async_copy(*gather_args(t, t % P))
    step(0, 0, do_drain=False, do_fire=True)

    def group(g, carry):
        for r in range(1, P + 1):
            t = P * g + r
            step(t, r % P, do_drain=True, do_fire=True)
        return carry

    lax.fori_loop(0, (STEPS - 4) // P, group, 0, unroll=False)

    for t in range(STEPS - 3, STEPS):
        step(t, t % P, do_drain=True, do_fire=(t + LOOKAHEAD < STEPS))
    for t in range(STEPS - DRAINLAG, STEPS):
        pltpu.make_async_copy(*store_args(t, t % P)).wait()


@jax.jit
def _embed_lookup(x1d, table):
    mesh = plsc.VectorSubcoreMesh(
        core_axis_name="c", subcore_axis_name="s", num_cores=NC, num_subcores=NS
    )
    run = pl.kernel(
        _gather_body,
        out_type=jax.ShapeDtypeStruct((N_FLAT, EMBED_DIM), jnp.float32),
        mesh=mesh,
        scratch_types=[
            pltpu.VMEM((ROWS_PER_W,), jnp.int32),
            pltpu.VMEM((P, STEP_ROWS, EMBED_DIM), jnp.float32),
            pltpu.SemaphoreType.DMA((P,)),
            pltpu.SemaphoreType.DMA((P,)),
        ],
    )
    return run(x1d, table)


def kernel(x, table):
    x1d = x.reshape(N_FLAT)
    out = _embed_lookup(x1d, table)
    return out.reshape(BATCH, HIST, EMBED_DIM)
